# baseline (speedup 1.0000x reference)
import functools

import jax
import jax.numpy as jnp
from jax import lax
from jax.experimental import pallas as pl
from jax.experimental.pallas import tpu as pltpu
from jax.experimental.pallas import tpu_sc as plsc

_NUM_EMBEDDINGS = 1000000
_MIN_SCALE = 2.5
_MAX_SCALE = 3.5

_ROWS = 16384
_COLS = 200
_TOTAL = _ROWS * _COLS

_info = plsc.get_sparse_core_info()
_NC = _info.num_cores
_NS = _info.num_subcores
_NW = _NC * _NS
_PER_W = _TOTAL // _NW

_CHUNK = 1024
_SEG = 128
_K = _CHUNK // _SEG
_STEPS = _PER_W // _CHUNK

_mesh = plsc.VectorSubcoreMesh(core_axis_name="c", subcore_axis_name="s")


@functools.partial(
    pl.kernel,
    mesh=_mesh,
    out_type=jax.ShapeDtypeStruct((_TOTAL,), jnp.float32),
    scratch_types=[
        pltpu.VMEM((_CHUNK,), jnp.int32),
        pltpu.VMEM((_CHUNK,), jnp.int32),
        pltpu.VMEM((_CHUNK,), jnp.float32),
        pltpu.VMEM((_CHUNK,), jnp.float32),
        pltpu.VMEM((_CHUNK,), jnp.float32),
        pltpu.VMEM((_CHUNK,), jnp.float32),
        pltpu.SemaphoreType.DMA,
    ],
)
def _sc_remap(ilo_hbm, ihi_hbm, frac_hbm, table_hbm, pin2d_hbm, out_hbm,
              ilo_v, ihi_v, frac_v, lo_v, hi_v, res_v, sem):
    del pin2d_hbm
    wid = lax.axis_index("s") * _NC + lax.axis_index("c")
    base = wid * _PER_W

    def step(i, carry):
        off = base + i * _CHUNK
        pltpu.sync_copy(ilo_hbm.at[pl.ds(off, _CHUNK)], ilo_v)
        pltpu.sync_copy(ihi_hbm.at[pl.ds(off, _CHUNK)], ihi_v)
        pltpu.sync_copy(frac_hbm.at[pl.ds(off, _CHUNK)], frac_v)

        copies = []
        for k in range(_K):
            sl = pl.ds(k * _SEG, _SEG)
            copies.append(pltpu.async_copy(
                table_hbm.at[ilo_v.at[sl]], lo_v.at[sl], sem))
            copies.append(pltpu.async_copy(
                table_hbm.at[ihi_v.at[sl]], hi_v.at[sl], sem))
        for cp in copies:
            cp.wait()

        def interp(j, c):
            sl = pl.ds(j * 16, 16)
            fr = frac_v[sl]
            res_v[sl] = fr * lo_v[sl] + (1.0 - fr) * hi_v[sl]
            return c

        lax.fori_loop(0, _CHUNK // 16, interp, 0, unroll=True)

        pltpu.sync_copy(res_v, out_hbm.at[pl.ds(off, _CHUNK)])
        return carry

    lax.fori_loop(0, _STEPS, step, 0)


def kernel(x, table, scale):
    s = jnp.clip(scale, _MIN_SCALE, _MAX_SCALE)
    mean = jnp.mean(x)
    std = jnp.std(x, ddof=1)
    out = (x - mean) / std
    out_01 = (jnp.clip(out, -s, s) / s + 1.0) / 2.0
    out3 = out_01 * (_NUM_EMBEDDINGS - 1)
    lower_1 = jnp.floor(out3)
    upper_1 = jnp.ceil(out3)
    lower_idx = lower_1.astype(jnp.int32)
    upper_idx = upper_1.astype(jnp.int32)
    frac = out3 - lower_1
    res = _sc_remap(lower_idx.reshape(-1), upper_idx.reshape(-1),
                    frac.reshape(-1), table.reshape(-1), out3)
    return res.reshape(_ROWS, _COLS)


# speedup vs baseline: 1.2956x; 1.2956x over previous
import functools

import jax
import jax.numpy as jnp
from jax import lax
from jax.experimental import pallas as pl
from jax.experimental.pallas import tpu as pltpu
from jax.experimental.pallas import tpu_sc as plsc

_NUM_EMBEDDINGS = 1000000
_MIN_SCALE = 2.5
_MAX_SCALE = 3.5

_ROWS = 16384
_COLS = 200
_TOTAL = _ROWS * _COLS

_info = plsc.get_sparse_core_info()
_NC = _info.num_cores
_NS = _info.num_subcores
_NW = _NC * _NS
_PER_W = _TOTAL // _NW

_CHUNK = 1024
_SEG = 128
_K = _CHUNK // _SEG
_STEPS = _PER_W // _CHUNK

_mesh = plsc.VectorSubcoreMesh(core_axis_name="c", subcore_axis_name="s")


@functools.partial(
    pl.kernel,
    mesh=_mesh,
    out_type=jax.ShapeDtypeStruct((_TOTAL,), jnp.float32),
    scratch_types=[
        pltpu.VMEM((_CHUNK,), jnp.float32),
        pltpu.VMEM((_CHUNK,), jnp.int32),
        pltpu.VMEM((_CHUNK,), jnp.int32),
        pltpu.VMEM((_CHUNK,), jnp.float32),
        pltpu.VMEM((_CHUNK,), jnp.float32),
        pltpu.SemaphoreType.DMA,
    ],
)
def _sc_remap(o3_hbm, table_hbm, pin2d_hbm, out_hbm,
              o3_v, ilo_v, ihi_v, lo_v, hi_v, sem):
    del pin2d_hbm
    wid = lax.axis_index("s") * _NC + lax.axis_index("c")
    base = wid * _PER_W

    def step(i, carry):
        off = base + i * _CHUNK
        pltpu.sync_copy(o3_hbm.at[pl.ds(off, _CHUNK)], o3_v)

        def derive(j, c):
            sl = pl.ds(j * 16, 16)
            o3 = o3_v[sl]
            li = o3.astype(jnp.int32)
            lf = li.astype(jnp.float32)
            fr = o3 - lf
            ilo_v[sl] = li
            ihi_v[sl] = li + jnp.where(fr > 0.0, 1, 0)
            o3_v[sl] = fr
            return c

        lax.fori_loop(0, _CHUNK // 16, derive, 0, unroll=True)

        copies = []
        for k in range(_K):
            sl = pl.ds(k * _SEG, _SEG)
            copies.append(pltpu.async_copy(
                table_hbm.at[ilo_v.at[sl]], lo_v.at[sl], sem))
            copies.append(pltpu.async_copy(
                table_hbm.at[ihi_v.at[sl]], hi_v.at[sl], sem))
        for cp in copies:
            cp.wait()

        def interp(j, c):
            sl = pl.ds(j * 16, 16)
            fr = o3_v[sl]
            o3_v[sl] = fr * lo_v[sl] + (1.0 - fr) * hi_v[sl]
            return c

        lax.fori_loop(0, _CHUNK // 16, interp, 0, unroll=True)

        pltpu.sync_copy(o3_v, out_hbm.at[pl.ds(off, _CHUNK)])
        return carry

    lax.fori_loop(0, _STEPS, step, 0)


def kernel(x, table, scale):
    s = jnp.clip(scale, _MIN_SCALE, _MAX_SCALE)
    mean = jnp.mean(x)
    std = jnp.std(x, ddof=1)
    out = (x - mean) / std
    out_01 = (jnp.clip(out, -s, s) / s + 1.0) / 2.0
    out3 = out_01 * (_NUM_EMBEDDINGS - 1)
    res = _sc_remap(out3.reshape(-1), table.reshape(-1), out3)
    return res.reshape(_ROWS, _COLS)


# speedup vs baseline: 1.4440x; 1.1146x over previous
import functools

import jax
import jax.numpy as jnp
from jax import lax
from jax.experimental import pallas as pl
from jax.experimental.pallas import tpu as pltpu
from jax.experimental.pallas import tpu_sc as plsc

_NUM_EMBEDDINGS = 1000000
_MIN_SCALE = 2.5
_MAX_SCALE = 3.5

_ROWS = 16384
_COLS = 200
_TOTAL = _ROWS * _COLS
_W = 128
_H = _TOTAL // _W

_info = plsc.get_sparse_core_info()
_NC = _info.num_cores
_NS = _info.num_subcores
_NW = _NC * _NS
_PER_W = _H // _NW

_R = 16
_STEPS = _PER_W // _R

_mesh = plsc.VectorSubcoreMesh(core_axis_name="c", subcore_axis_name="s")


@functools.partial(
    pl.kernel,
    mesh=_mesh,
    out_type=jax.ShapeDtypeStruct((_H, _W), jnp.float32),
    scratch_types=[
        pltpu.VMEM((_R, _W), jnp.float32),
        pltpu.VMEM((_R, _W), jnp.int32),
        pltpu.VMEM((_R, _W), jnp.int32),
        pltpu.VMEM((_R, _W), jnp.float32),
        pltpu.VMEM((_R, _W), jnp.float32),
        pltpu.SemaphoreType.DMA,
    ],
)
def _sc_remap(o3_hbm, table_hbm, pin2d_hbm, out_hbm,
              o3_v, ilo_v, ihi_v, lo_v, hi_v, sem):
    del pin2d_hbm
    wid = lax.axis_index("s") * _NC + lax.axis_index("c")
    base = wid * _PER_W

    def step(i, carry):
        row0 = base + i * _R
        pltpu.sync_copy(o3_hbm.at[pl.ds(row0, _R), :], o3_v)

        def derive(j, c):
            r = j // 8
            sl = pl.ds((j % 8) * 16, 16)
            o3 = o3_v[r, sl]
            li = o3.astype(jnp.int32)
            lf = li.astype(jnp.float32)
            fr = o3 - lf
            ilo_v[r, sl] = li
            ihi_v[r, sl] = li + jnp.where(fr > 0.0, 1, 0)
            o3_v[r, sl] = fr
            return c

        lax.fori_loop(0, _R * 8, derive, 0, unroll=True)

        copies = []
        for r in range(_R):
            copies.append(pltpu.async_copy(
                table_hbm.at[ilo_v.at[r]], lo_v.at[r], sem))
            copies.append(pltpu.async_copy(
                table_hbm.at[ihi_v.at[r]], hi_v.at[r], sem))
        for cp in copies:
            cp.wait()

        def interp(j, c):
            r = j // 8
            sl = pl.ds((j % 8) * 16, 16)
            fr = o3_v[r, sl]
            o3_v[r, sl] = fr * lo_v[r, sl] + (1.0 - fr) * hi_v[r, sl]
            return c

        lax.fori_loop(0, _R * 8, interp, 0, unroll=True)

        pltpu.sync_copy(o3_v, out_hbm.at[pl.ds(row0, _R), :])
        return carry

    lax.fori_loop(0, _STEPS, step, 0)


def kernel(x, table, scale):
    s = jnp.clip(scale, _MIN_SCALE, _MAX_SCALE)
    mean = jnp.mean(x)
    std = jnp.std(x, ddof=1)
    out = (x - mean) / std
    out_01 = (jnp.clip(out, -s, s) / s + 1.0) / 2.0
    out3 = out_01 * (_NUM_EMBEDDINGS - 1)
    res = _sc_remap(out3.reshape(_H, _W), table.reshape(-1), out3)
    return res.reshape(_ROWS, _COLS)


# speedup vs baseline: 1.6959x; 1.1744x over previous
import functools

import jax
import jax.numpy as jnp
from jax import lax
from jax.experimental import pallas as pl
from jax.experimental.pallas import tpu as pltpu
from jax.experimental.pallas import tpu_sc as plsc

_NUM_EMBEDDINGS = 1000000
_MIN_SCALE = 2.5
_MAX_SCALE = 3.5

_ROWS = 16384
_COLS = 200
_TOTAL = _ROWS * _COLS
_W = 128
_H = _TOTAL // _W

_info = plsc.get_sparse_core_info()
_NC = _info.num_cores
_NS = _info.num_subcores
_NW = _NC * _NS
_PER_W = _H // _NW

_R = 8
_STEPS = _PER_W // _R
_OUTER = _STEPS // 2

_mesh = plsc.VectorSubcoreMesh(core_axis_name="c", subcore_axis_name="s")


@functools.partial(
    pl.kernel,
    mesh=_mesh,
    out_type=jax.ShapeDtypeStruct((_H, _W), jnp.float32),
    scratch_types=(
        [pltpu.VMEM((_R, _W), jnp.float32) for _ in range(2)]
        + [pltpu.VMEM((_R, _W), jnp.int32) for _ in range(2)]
        + [pltpu.VMEM((_R, _W), jnp.int32) for _ in range(2)]
        + [pltpu.VMEM((_R, _W), jnp.float32) for _ in range(2)]
        + [pltpu.VMEM((_R, _W), jnp.float32) for _ in range(2)]
        + [pltpu.SemaphoreType.DMA, pltpu.SemaphoreType.DMA,
           pltpu.SemaphoreType.DMA, pltpu.SemaphoreType.DMA]
    ),
)
def _sc_remap(o3_hbm, table_hbm, pin2d_hbm, out_hbm,
              o3_0, o3_1, ilo_0, ilo_1, ihi_0, ihi_1,
              lo_0, lo_1, hi_0, hi_1, sem_g0, sem_g1, sem_o0, sem_o1):
    del pin2d_hbm
    o3_v = (o3_0, o3_1)
    ilo_v = (ilo_0, ilo_1)
    ihi_v = (ihi_0, ihi_1)
    lo_v = (lo_0, lo_1)
    hi_v = (hi_0, hi_1)
    sem_g = (sem_g0, sem_g1)
    sem_o = (sem_o0, sem_o1)

    wid = lax.axis_index("s") * _NC + lax.axis_index("c")
    base = wid * _PER_W

    def rows(i):
        return pl.ds(base + i * _R, _R)

    def derive(b):
        def body(j, c):
            r = j // 8
            sl = pl.ds((j % 8) * 16, 16)
            o3 = o3_v[b][r, sl]
            li = o3.astype(jnp.int32)
            lf = li.astype(jnp.float32)
            fr = o3 - lf
            ilo_v[b][r, sl] = li
            ihi_v[b][r, sl] = li + jnp.where(fr > 0.0, 1, 0)
            o3_v[b][r, sl] = fr
            return c
        lax.fori_loop(0, _R * 8, body, 0, unroll=True)

    def fire(b):
        for r in range(_R):
            pltpu.async_copy(table_hbm.at[ilo_v[b].at[r]], lo_v[b].at[r],
                             sem_g[b])
            pltpu.async_copy(table_hbm.at[ihi_v[b].at[r]], hi_v[b].at[r],
                             sem_g[b])

    def drain_gathers(i, b):
        pltpu.make_async_copy(o3_hbm.at[rows(i), :], lo_v[b], sem_g[b]).wait()
        pltpu.make_async_copy(o3_hbm.at[rows(i), :], hi_v[b], sem_g[b]).wait()

    def interp(b):
        def body(j, c):
            r = j // 8
            sl = pl.ds((j % 8) * 16, 16)
            fr = o3_v[b][r, sl]
            o3_v[b][r, sl] = fr * lo_v[b][r, sl] + (1.0 - fr) * hi_v[b][r, sl]
            return c
        lax.fori_loop(0, _R * 8, body, 0, unroll=True)

    def outer(g, carry):
        for b in range(2):
            i = 2 * g + b

            @pl.when(i >= 2)
            def _():
                pltpu.make_async_copy(
                    o3_hbm.at[rows(i - 2), :], o3_v[b], sem_o[b]).wait()

            pltpu.sync_copy(o3_hbm.at[rows(i), :], o3_v[b])
            derive(b)
            fire(b)

            @pl.when(i >= 1)
            def _():
                pb = 1 - b
                drain_gathers(i - 1, pb)
                interp(pb)
                pltpu.async_copy(o3_v[pb], out_hbm.at[rows(i - 1), :],
                                 sem_o[pb])

        return carry

    lax.fori_loop(0, _OUTER, outer, 0)

    last = _STEPS - 1
    lb = last % 2
    drain_gathers(last, lb)
    interp(lb)
    pltpu.sync_copy(o3_v[lb], out_hbm.at[rows(last), :])
    pltpu.make_async_copy(o3_hbm.at[rows(last - 1), :], o3_v[1 - lb],
                          sem_o[1 - lb]).wait()


def kernel(x, table, scale):
    s = jnp.clip(scale, _MIN_SCALE, _MAX_SCALE)
    mean = jnp.mean(x)
    std = jnp.std(x, ddof=1)
    out = (x - mean) / std
    out_01 = (jnp.clip(out, -s, s) / s + 1.0) / 2.0
    out3 = out_01 * (_NUM_EMBEDDINGS - 1)
    res = _sc_remap(out3.reshape(_H, _W), table.reshape(-1), out3)
    return res.reshape(_ROWS, _COLS)
